# speedup vs baseline: 1.3818x; 1.3818x over previous
import functools

import jax
import jax.numpy as jnp
from jax import lax
from jax.experimental import pallas as pl
from jax.experimental.pallas import tpu as pltpu

TOP_K = 8
BM = 128
MAX_STEPS = 192
PN = MAX_STEPS * BM


def _router_shared_body(x_ref, gate_ref, w1_ref, w2_ref, w3_ref,
                        shared_ref, sel_ref, tw_ref):
    x = x_ref[...]
    a = lax.dot_general(x, w1_ref[...], (((1,), (1,)), ((), ())),
                        preferred_element_type=jnp.float32)
    b = lax.dot_general(x, w3_ref[...], (((1,), (1,)), ((), ())),
                        preferred_element_type=jnp.float32)
    h = a * jax.nn.sigmoid(a) * b
    shared_ref[...] = lax.dot_general(h, w2_ref[...], (((1,), (1,)), ((), ())),
                                      preferred_element_type=jnp.float32)
    logits = lax.dot_general(x, gate_ref[...], (((1,), (1,)), ((), ())),
                             preferred_element_type=jnp.float32)
    m = jnp.max(logits, axis=1, keepdims=True)
    p = jnp.exp(logits - m)
    probs = p / jnp.sum(p, axis=1, keepdims=True)
    ncols = probs.shape[1]
    iota = lax.broadcasted_iota(jnp.int32, probs.shape, 1)
    cur = probs
    for k in range(TOP_K):
        mk = jnp.max(cur, axis=1, keepdims=True)
        idx = jnp.min(jnp.where(cur == mk, iota, ncols), axis=1, keepdims=True)
        sel_ref[:, k:k + 1] = idx
        tw_ref[:, k:k + 1] = mk
        cur = jnp.where(iota == idx, -jnp.inf, cur)


def _router_shared(x, gate_w, sw1, sw2, sw3, interpret=False):
    T, H = x.shape
    E = gate_w.shape[0]
    SI = sw1.shape[0]
    BT = 256
    grid = (T // BT,)
    return pl.pallas_call(
        _router_shared_body,
        grid=grid,
        in_specs=[
            pl.BlockSpec((BT, H), lambda i: (i, 0)),
            pl.BlockSpec((E, H), lambda i: (0, 0)),
            pl.BlockSpec((SI, H), lambda i: (0, 0)),
            pl.BlockSpec((H, SI), lambda i: (0, 0)),
            pl.BlockSpec((SI, H), lambda i: (0, 0)),
        ],
        out_specs=[
            pl.BlockSpec((BT, H), lambda i: (i, 0)),
            pl.BlockSpec((BT, TOP_K), lambda i: (i, 0)),
            pl.BlockSpec((BT, TOP_K), lambda i: (i, 0)),
        ],
        out_shape=[
            jax.ShapeDtypeStruct((T, H), jnp.float32),
            jax.ShapeDtypeStruct((T, TOP_K), jnp.int32),
            jax.ShapeDtypeStruct((T, TOP_K), jnp.float32),
        ],
        interpret=interpret,
    )(x, gate_w, sw1, sw2, sw3)


def _dispatch(x, sel, tw):
    T, H = x.shape
    E = 64
    N = T * TOP_K
    slot_e = sel.reshape(-1)
    slot_t = lax.broadcasted_iota(jnp.int32, (T, TOP_K), 0).reshape(-1)
    counts = jnp.bincount(slot_e, length=E)
    padded = ((counts + BM - 1) // BM) * BM
    base = jnp.concatenate([jnp.zeros(1, jnp.int32),
                            jnp.cumsum(padded)[:-1].astype(jnp.int32)])
    grp_start = jnp.concatenate([jnp.zeros(1, jnp.int32),
                                 jnp.cumsum(counts)[:-1].astype(jnp.int32)])
    order = jnp.argsort(slot_e, stable=True)
    e_sorted = slot_e[order]
    within = jnp.arange(N, dtype=jnp.int32) - grp_start[e_sorted]
    pos_sorted = base[e_sorted] + within
    x_sorted = jnp.zeros((PN, H), x.dtype).at[pos_sorted].set(x[slot_t[order]])
    w_sorted = jnp.zeros((PN, 1), x.dtype).at[pos_sorted, 0].set(tw.reshape(-1)[order])
    posmap = jnp.zeros((N,), jnp.int32).at[order].set(pos_sorted).reshape(T, TOP_K)
    n_tiles = padded // BM
    nsteps = jnp.sum(n_tiles)
    csteps = jnp.concatenate([jnp.zeros(1, jnp.int32),
                              jnp.cumsum(n_tiles)[:-1].astype(jnp.int32)])
    s = jnp.arange(MAX_STEPS, dtype=jnp.int32)
    active = (s < nsteps).astype(jnp.int32)
    sc = jnp.minimum(s, nsteps - 1)
    step_expert = jnp.searchsorted(jnp.cumsum(n_tiles), sc, side='right').astype(jnp.int32)
    step_rowtile = (base[step_expert] // BM + (sc - csteps[step_expert])).astype(jnp.int32)
    return x_sorted, w_sorted, posmap, step_expert, step_rowtile, active


def _gemm_body(se_ref, sr_ref, act_ref, x_ref, w1_ref, w2_ref, w3_ref, ws_ref, y_ref):
    s = pl.program_id(0)

    @pl.when(act_ref[s] == 1)
    def _():
        x = x_ref[...]
        a = lax.dot_general(x, w1_ref[0], (((1,), (1,)), ((), ())),
                            preferred_element_type=jnp.float32)
        b = lax.dot_general(x, w3_ref[0], (((1,), (1,)), ((), ())),
                            preferred_element_type=jnp.float32)
        h = a * jax.nn.sigmoid(a) * b * ws_ref[...]
        y_ref[...] = lax.dot_general(h, w2_ref[0], (((1,), (1,)), ((), ())),
                                     preferred_element_type=jnp.float32)


def _grouped_gemm(x_sorted, w_sorted, ew1, ew2, ew3,
                  step_expert, step_rowtile, active, interpret=False):
    E, I, H = ew1.shape
    grid_spec = pltpu.PrefetchScalarGridSpec(
        num_scalar_prefetch=3,
        grid=(MAX_STEPS,),
        in_specs=[
            pl.BlockSpec((BM, H), lambda s, se, sr, act: (sr[s], 0)),
            pl.BlockSpec((1, I, H), lambda s, se, sr, act: (se[s], 0, 0)),
            pl.BlockSpec((1, H, I), lambda s, se, sr, act: (se[s], 0, 0)),
            pl.BlockSpec((1, I, H), lambda s, se, sr, act: (se[s], 0, 0)),
            pl.BlockSpec((BM, 1), lambda s, se, sr, act: (sr[s], 0)),
        ],
        out_specs=pl.BlockSpec((BM, H), lambda s, se, sr, act: (sr[s], 0)),
    )
    return pl.pallas_call(
        _gemm_body,
        grid_spec=grid_spec,
        out_shape=jax.ShapeDtypeStruct((PN, H), jnp.float32),
        interpret=interpret,
    )(step_expert, step_rowtile, active, x_sorted, ew1, ew2, ew3, w_sorted)


def _combine(shared_out, y_sorted, posmap):
    return shared_out + jnp.sum(y_sorted[posmap], axis=1)


def _moe(hidden_states, gate_w, sw1, sw2, sw3, ew1, ew2, ew3, interpret=False):
    B, S, H = hidden_states.shape
    x = hidden_states.reshape(-1, H)
    shared_out, sel, tw = _router_shared(x, gate_w, sw1, sw2, sw3, interpret=interpret)
    x_sorted, w_sorted, posmap, step_expert, step_rowtile, active = _dispatch(x, sel, tw)
    y_sorted = _grouped_gemm(x_sorted, w_sorted, ew1, ew2, ew3,
                             step_expert, step_rowtile, active, interpret=interpret)
    final = _combine(shared_out, y_sorted, posmap)
    return final.reshape(B, S, H)


def kernel(hidden_states, gate_w, shared_w1, shared_w2, shared_w3,
           expert_w1, expert_w2, expert_w3):
    return _moe(hidden_states, gate_w, shared_w1, shared_w2, shared_w3,
                expert_w1, expert_w2, expert_w3)
